# Initial kernel scaffold; baseline (speedup 1.0000x reference)
import jax
import jax.numpy as jnp
from jax.experimental import pallas as pl


def kernel(X, enc_W1, enc_b1, enc_W2, enc_b2, Ws, Wr, B, g_W1, g_b1, g_W2, g_b2, g_Wo, g_bo):
    raise NotImplementedError("write your pallas kernel here")


# speedup vs baseline: 9.6904x; 9.6904x over previous
import numpy as np
import jax
import jax.numpy as jnp
from jax.experimental import pallas as pl

N = 4096
D_IN = 128
RANK = 16
D_HIDDEN = 128
BOTTLENECK = 64
TOPK = 32

PRE_BLK = 512
BLK = 256

_SIGN = np.int32(-2**31)
_MANT = np.int32(0x7FFFFFFF)


def _keys(l):
    bits = jax.lax.bitcast_convert_type(l, jnp.int32)
    return jnp.where(bits < 0, bits ^ _MANT, bits)


def _unkey(k):
    bits = jnp.where(k < 0, k ^ _MANT, k)
    return jax.lax.bitcast_convert_type(bits, jnp.float32)


def _prelude_kernel(x_ref, ew1_ref, eb1_ref, ew2_ref, eb2_ref, ws_ref, wr_ref,
                    b_ref, xm_ref, sb_ref, r_ref):
    x = x_ref[...]
    h = jnp.maximum(
        jnp.dot(x, ew1_ref[...], preferred_element_type=jnp.float32)
        + eb1_ref[...], 0.0)
    xm = x + jnp.dot(h, ew2_ref[...], preferred_element_type=jnp.float32) \
        + eb2_ref[...]
    xm_ref[...] = xm
    s = jax.nn.softmax(
        jnp.dot(xm, ws_ref[...], preferred_element_type=jnp.float32), axis=-1)
    r = jax.nn.softmax(
        jnp.dot(xm, wr_ref[...], preferred_element_type=jnp.float32), axis=-1)
    sb_ref[...] = jnp.dot(s, b_ref[...], preferred_element_type=jnp.float32)
    r_ref[...] = r


def _round1_kernel(sb_ref, r_ref, xm_ref, gw1_ref, gb1_ref,
                   h1_ref, thr_ref, shift_ref):
    sb = sb_ref[...]
    r = r_ref[...]
    l = jax.lax.dot_general(sb, r, (((1,), (1,)), ((), ())),
                            preferred_element_type=jnp.float32)
    ki = _keys(l)

    def body(i, prefix):
        bit = jax.lax.shift_left(jnp.int32(1), jnp.int32(31) - i)
        cand_b = prefix | bit
        cand_i = cand_b ^ _SIGN
        cnt = jnp.sum((ki >= cand_i).astype(jnp.int32), axis=1, keepdims=True)
        return jnp.where(cnt >= TOPK, cand_b, prefix)

    prefix = jax.lax.fori_loop(0, 32, body, jnp.zeros((BLK, 1), jnp.int32))
    thr_f = _unkey(prefix ^ _SIGN)

    m = jnp.max(l, axis=1, keepdims=True)
    p = jnp.where(l >= thr_f, jnp.exp(l - m), 0.0)
    denom = jnp.sum(p, axis=1, keepdims=True)
    w = p * (1.0 / denom)
    msg = jnp.dot(w, xm_ref[...], preferred_element_type=jnp.float32)
    h1 = jnp.maximum(
        jnp.dot(msg, gw1_ref[...], preferred_element_type=jnp.float32)
        + gb1_ref[...], 0.0)
    h1_ref[...] = h1
    thr_ref[...] = thr_f
    shift_ref[...] = m + jnp.log(denom)


def _round2_kernel(sb_ref, r_ref, h1_ref, thr_ref, shift_ref,
                   gw2_ref, gb2_ref, gwo_ref, gbo_ref, y_ref):
    sb = sb_ref[...]
    r = r_ref[...]
    l = jax.lax.dot_general(sb, r, (((1,), (1,)), ((), ())),
                            preferred_element_type=jnp.float32)
    w = jnp.where(l >= thr_ref[...], jnp.exp(l - shift_ref[...]), 0.0)
    msg = jnp.dot(w, h1_ref[...], preferred_element_type=jnp.float32)
    h2 = jnp.maximum(
        jnp.dot(msg, gw2_ref[...], preferred_element_type=jnp.float32)
        + gb2_ref[...], 0.0)
    y_ref[...] = jnp.dot(h2, gwo_ref[...], preferred_element_type=jnp.float32) \
        + gbo_ref[...]


def kernel(X, enc_W1, enc_b1, enc_W2, enc_b2, Ws, Wr, B,
           g_W1, g_b1, g_W2, g_b2, g_Wo, g_bo):
    eb1 = enc_b1.reshape(1, BOTTLENECK)
    eb2 = enc_b2.reshape(1, D_IN)
    gb1 = g_b1.reshape(1, D_HIDDEN)
    gb2 = g_b2.reshape(1, D_HIDDEN)
    gbo = g_bo.reshape(1, 1)

    full = lambda shape: pl.BlockSpec(shape, lambda i: (0, 0))

    xm, sb, r = pl.pallas_call(
        _prelude_kernel,
        grid=(N // PRE_BLK,),
        in_specs=[
            pl.BlockSpec((PRE_BLK, D_IN), lambda i: (i, 0)),
            full((D_IN, BOTTLENECK)), full((1, BOTTLENECK)),
            full((BOTTLENECK, D_IN)), full((1, D_IN)),
            full((D_IN, RANK)), full((D_IN, RANK)), full((RANK, RANK)),
        ],
        out_specs=[
            pl.BlockSpec((PRE_BLK, D_IN), lambda i: (i, 0)),
            pl.BlockSpec((PRE_BLK, RANK), lambda i: (i, 0)),
            pl.BlockSpec((PRE_BLK, RANK), lambda i: (i, 0)),
        ],
        out_shape=[
            jax.ShapeDtypeStruct((N, D_IN), jnp.float32),
            jax.ShapeDtypeStruct((N, RANK), jnp.float32),
            jax.ShapeDtypeStruct((N, RANK), jnp.float32),
        ],
    )(X, enc_W1, eb1, enc_W2, eb2, Ws, Wr, B)

    h1, thr, shift = pl.pallas_call(
        _round1_kernel,
        grid=(N // BLK,),
        in_specs=[
            pl.BlockSpec((BLK, RANK), lambda i: (i, 0)),
            full((N, RANK)), full((N, D_IN)),
            full((D_IN, D_HIDDEN)), full((1, D_HIDDEN)),
        ],
        out_specs=[
            pl.BlockSpec((BLK, D_HIDDEN), lambda i: (i, 0)),
            pl.BlockSpec((BLK, 1), lambda i: (i, 0)),
            pl.BlockSpec((BLK, 1), lambda i: (i, 0)),
        ],
        out_shape=[
            jax.ShapeDtypeStruct((N, D_HIDDEN), jnp.float32),
            jax.ShapeDtypeStruct((N, 1), jnp.float32),
            jax.ShapeDtypeStruct((N, 1), jnp.float32),
        ],
    )(sb, r, xm, g_W1, gb1)

    y = pl.pallas_call(
        _round2_kernel,
        grid=(N // BLK,),
        in_specs=[
            pl.BlockSpec((BLK, RANK), lambda i: (i, 0)),
            full((N, RANK)), full((N, D_HIDDEN)),
            pl.BlockSpec((BLK, 1), lambda i: (i, 0)),
            pl.BlockSpec((BLK, 1), lambda i: (i, 0)),
            full((D_HIDDEN, D_HIDDEN)), full((1, D_HIDDEN)),
            full((D_HIDDEN, 1)), full((1, 1)),
        ],
        out_specs=pl.BlockSpec((BLK, 1), lambda i: (i, 0)),
        out_shape=jax.ShapeDtypeStruct((N, 1), jnp.float32),
    )(sb, r, h1, thr, shift, g_W2, gb2, g_Wo, gbo)

    return y


# speedup vs baseline: 10.0244x; 1.0345x over previous
import numpy as np
import jax
import jax.numpy as jnp
from jax.experimental import pallas as pl

N = 4096
D_IN = 128
RANK = 16
D_HIDDEN = 128
BOTTLENECK = 64
TOPK = 32

PRE_BLK = 512
BLK = 256

_SIGN = np.int32(-2**31)
_MANT = np.int32(0x7FFFFFFF)


def _keys(l):
    bits = jax.lax.bitcast_convert_type(l, jnp.int32)
    return jnp.where(bits < 0, bits ^ _MANT, bits)


def _unkey(k):
    bits = jnp.where(k < 0, k ^ _MANT, k)
    return jax.lax.bitcast_convert_type(bits, jnp.float32)


def _prelude_kernel(x_ref, ew1_ref, eb1_ref, ew2_ref, eb2_ref, ws_ref, wr_ref,
                    b_ref, xm_ref, sb_ref, r_ref):
    x = x_ref[...]
    h = jnp.maximum(
        jnp.dot(x, ew1_ref[...], preferred_element_type=jnp.float32)
        + eb1_ref[...], 0.0)
    xm = x + jnp.dot(h, ew2_ref[...], preferred_element_type=jnp.float32) \
        + eb2_ref[...]
    xm_ref[...] = xm
    s = jax.nn.softmax(
        jnp.dot(xm, ws_ref[...], preferred_element_type=jnp.float32), axis=-1)
    r = jax.nn.softmax(
        jnp.dot(xm, wr_ref[...], preferred_element_type=jnp.float32), axis=-1)
    sb_ref[...] = jnp.dot(s, b_ref[...], preferred_element_type=jnp.float32)
    r_ref[...] = r


def _kth_key(keys, k, lo, hi):
    def cond(carry):
        lo_c, hi_c = carry
        return jnp.any(hi_c > lo_c)

    def body(carry):
        lo_c, hi_c = carry
        c = lo_c + jax.lax.shift_right_logical((hi_c - lo_c) + 1, 1)
        cnt = jnp.sum((keys >= c).astype(jnp.int32), axis=1, keepdims=True)
        feas = cnt >= k
        return jnp.where(feas, c, lo_c), jnp.where(feas, hi_c, c - 1)

    lo, hi = jax.lax.while_loop(cond, body, (lo, hi))
    return lo


def _round1_kernel(sb_ref, r_ref, xm_ref, gw1_ref, gb1_ref,
                   h1_ref, thr_ref, shift_ref):
    sb = sb_ref[...]
    r = r_ref[...]
    l = jax.lax.dot_general(sb, r, (((1,), (1,)), ((), ())),
                            preferred_element_type=jnp.float32)
    ki = _keys(l)

    kv = ki[:, 0:128]
    for c in range(1, N // 128):
        kv = jnp.maximum(kv, ki[:, c * 128:(c + 1) * 128])
    lo1 = jnp.min(kv, axis=1, keepdims=True)
    hi1 = jnp.max(kv, axis=1, keepdims=True)
    t0 = _kth_key(kv, TOPK, lo1, hi1)
    t_star = _kth_key(ki, TOPK, t0, hi1)
    thr_f = _unkey(t_star)

    m = jnp.max(l, axis=1, keepdims=True)
    p = jnp.where(l >= thr_f, jnp.exp(l - m), 0.0)
    denom = jnp.sum(p, axis=1, keepdims=True)
    w = p * (1.0 / denom)
    msg = jnp.dot(w, xm_ref[...], preferred_element_type=jnp.float32)
    h1 = jnp.maximum(
        jnp.dot(msg, gw1_ref[...], preferred_element_type=jnp.float32)
        + gb1_ref[...], 0.0)
    h1_ref[...] = h1
    thr_ref[...] = thr_f
    shift_ref[...] = m + jnp.log(denom)


def _round2_kernel(sb_ref, r_ref, h1_ref, thr_ref, shift_ref,
                   gw2_ref, gb2_ref, gwo_ref, gbo_ref, y_ref):
    sb = sb_ref[...]
    r = r_ref[...]
    l = jax.lax.dot_general(sb, r, (((1,), (1,)), ((), ())),
                            preferred_element_type=jnp.float32)
    w = jnp.where(l >= thr_ref[...], jnp.exp(l - shift_ref[...]), 0.0)
    msg = jnp.dot(w, h1_ref[...], preferred_element_type=jnp.float32)
    h2 = jnp.maximum(
        jnp.dot(msg, gw2_ref[...], preferred_element_type=jnp.float32)
        + gb2_ref[...], 0.0)
    y_ref[...] = jnp.dot(h2, gwo_ref[...], preferred_element_type=jnp.float32) \
        + gbo_ref[...]


def kernel(X, enc_W1, enc_b1, enc_W2, enc_b2, Ws, Wr, B,
           g_W1, g_b1, g_W2, g_b2, g_Wo, g_bo):
    eb1 = enc_b1.reshape(1, BOTTLENECK)
    eb2 = enc_b2.reshape(1, D_IN)
    gb1 = g_b1.reshape(1, D_HIDDEN)
    gb2 = g_b2.reshape(1, D_HIDDEN)
    gbo = g_bo.reshape(1, 1)

    full = lambda shape: pl.BlockSpec(shape, lambda i: (0, 0))

    xm, sb, r = pl.pallas_call(
        _prelude_kernel,
        grid=(N // PRE_BLK,),
        in_specs=[
            pl.BlockSpec((PRE_BLK, D_IN), lambda i: (i, 0)),
            full((D_IN, BOTTLENECK)), full((1, BOTTLENECK)),
            full((BOTTLENECK, D_IN)), full((1, D_IN)),
            full((D_IN, RANK)), full((D_IN, RANK)), full((RANK, RANK)),
        ],
        out_specs=[
            pl.BlockSpec((PRE_BLK, D_IN), lambda i: (i, 0)),
            pl.BlockSpec((PRE_BLK, RANK), lambda i: (i, 0)),
            pl.BlockSpec((PRE_BLK, RANK), lambda i: (i, 0)),
        ],
        out_shape=[
            jax.ShapeDtypeStruct((N, D_IN), jnp.float32),
            jax.ShapeDtypeStruct((N, RANK), jnp.float32),
            jax.ShapeDtypeStruct((N, RANK), jnp.float32),
        ],
    )(X, enc_W1, eb1, enc_W2, eb2, Ws, Wr, B)

    h1, thr, shift = pl.pallas_call(
        _round1_kernel,
        grid=(N // BLK,),
        in_specs=[
            pl.BlockSpec((BLK, RANK), lambda i: (i, 0)),
            full((N, RANK)), full((N, D_IN)),
            full((D_IN, D_HIDDEN)), full((1, D_HIDDEN)),
        ],
        out_specs=[
            pl.BlockSpec((BLK, D_HIDDEN), lambda i: (i, 0)),
            pl.BlockSpec((BLK, 1), lambda i: (i, 0)),
            pl.BlockSpec((BLK, 1), lambda i: (i, 0)),
        ],
        out_shape=[
            jax.ShapeDtypeStruct((N, D_HIDDEN), jnp.float32),
            jax.ShapeDtypeStruct((N, 1), jnp.float32),
            jax.ShapeDtypeStruct((N, 1), jnp.float32),
        ],
    )(sb, r, xm, g_W1, gb1)

    y = pl.pallas_call(
        _round2_kernel,
        grid=(N // BLK,),
        in_specs=[
            pl.BlockSpec((BLK, RANK), lambda i: (i, 0)),
            full((N, RANK)), full((N, D_HIDDEN)),
            pl.BlockSpec((BLK, 1), lambda i: (i, 0)),
            pl.BlockSpec((BLK, 1), lambda i: (i, 0)),
            full((D_HIDDEN, D_HIDDEN)), full((1, D_HIDDEN)),
            full((D_HIDDEN, 1)), full((1, 1)),
        ],
        out_specs=pl.BlockSpec((BLK, 1), lambda i: (i, 0)),
        out_shape=jax.ShapeDtypeStruct((N, 1), jnp.float32),
    )(sb, r, h1, thr, shift, g_W2, gb2, g_Wo, gbo)

    return y
